# Initial kernel scaffold; baseline (speedup 1.0000x reference)
import functools

import jax
import jax.numpy as jnp
from jax import lax
from jax.experimental import pallas as pl
from jax.experimental.pallas import tpu as pltpu
from jax.experimental.pallas import tpu_sc as plsc

B, L, D = 4096, 200, 64
NC, NS = 2, 16
NW = NC * NS
SEQ_PER_W = B // NW
S0, S1 = 96, 104
LANES = 16


def _encoder_sc(xf, wt, pt):
    mesh = plsc.VectorSubcoreMesh(core_axis_name="c", subcore_axis_name="s")

    @functools.partial(
        pl.kernel,
        mesh=mesh,
        out_type=jax.ShapeDtypeStruct((B, L, D), jnp.float32),
        scratch_types=[
            pltpu.VMEM((L,), jnp.int32),
            pltpu.VMEM((L, D), jnp.float32),
            pltpu.VMEM((L, D), jnp.float32),
            pltpu.SemaphoreType.DMA,
        ],
    )
    def k(xf_hbm, wt_hbm, pt_hbm, out_hbm, idx_v, rows_v, pos_v, gsem):
        wid = lax.axis_index("s") * NC + lax.axis_index("c")
        pltpu.sync_copy(pt_hbm.at[pl.ds(0, L)], pos_v)

        def chunk(g, carry):
            seq = wid * SEQ_PER_W + g
            pltpu.sync_copy(xf_hbm.at[pl.ds(seq * L, L)], idx_v)
            c1 = pltpu.async_copy(wt_hbm.at[idx_v.at[pl.ds(0, S0)]],
                                  rows_v.at[pl.ds(0, S0)], gsem)
            c2 = pltpu.async_copy(wt_hbm.at[idx_v.at[pl.ds(S0, S1)]],
                                  rows_v.at[pl.ds(S0, S1)], gsem)
            c1.wait()
            c2.wait()

            def add_row(r, rcarry):
                for c in range(D // LANES):
                    sl = pl.ds(c * LANES, LANES)
                    rows_v[r, sl] = rows_v[r, sl] + pos_v[r, sl]
                return rcarry

            lax.fori_loop(0, L, add_row, 0)
            pltpu.sync_copy(rows_v, out_hbm.at[seq])
            return carry

        lax.fori_loop(0, SEQ_PER_W, chunk, 0)

    return k(xf, wt, pt)


def kernel(x, word_table, pos_table):
    xf = x.reshape(-1).astype(jnp.int32)
    return _encoder_sc(xf, word_table, pos_table)


# speedup vs baseline: 3.1050x; 3.1050x over previous
import functools

import jax
import jax.numpy as jnp
from jax import lax
from jax.experimental import pallas as pl
from jax.experimental.pallas import tpu as pltpu
from jax.experimental.pallas import tpu_sc as plsc

B, L, D = 4096, 200, 64
NC, NS = 2, 16
NW = NC * NS
SEQ_PER_W = B // NW
S0, S1 = 96, 104
LANES = 16


def _encoder_sc(xf, wt, pt):
    mesh = plsc.VectorSubcoreMesh(core_axis_name="c", subcore_axis_name="s")

    @functools.partial(
        pl.kernel,
        mesh=mesh,
        out_type=jax.ShapeDtypeStruct((B, L, D), jnp.float32),
        scratch_types=[
            pltpu.VMEM((L,), jnp.int32),
            pltpu.VMEM((L, D), jnp.float32),
            pltpu.VMEM((L, D), jnp.float32),
            pltpu.SemaphoreType.DMA,
        ],
        compiler_params=pltpu.CompilerParams(use_tc_tiling_on_sc=False),
    )
    def k(xf_hbm, wt_hbm, pt_hbm, out_hbm, idx_v, rows_v, pos_v, gsem):
        wid = lax.axis_index("s") * NC + lax.axis_index("c")
        pltpu.sync_copy(pt_hbm.at[pl.ds(0, L)], pos_v)

        def chunk(g, carry):
            seq = wid * SEQ_PER_W + g
            pltpu.sync_copy(xf_hbm.at[pl.ds(seq * L, L)], idx_v)
            c1 = pltpu.async_copy(wt_hbm.at[idx_v.at[pl.ds(0, S0)]],
                                  rows_v.at[pl.ds(0, S0)], gsem)
            c2 = pltpu.async_copy(wt_hbm.at[idx_v.at[pl.ds(S0, S1)]],
                                  rows_v.at[pl.ds(S0, S1)], gsem)
            c1.wait()
            c2.wait()

            def add_row(r, rcarry):
                for c in range(D // LANES):
                    sl = pl.ds(c * LANES, LANES)
                    rows_v[r, sl] = rows_v[r, sl] + pos_v[r, sl]
                return rcarry

            lax.fori_loop(0, L, add_row, 0)
            pltpu.sync_copy(rows_v, out_hbm.at[seq])
            return carry

        lax.fori_loop(0, SEQ_PER_W, chunk, 0)

    return k(xf, wt, pt)


def kernel(x, word_table, pos_table):
    xf = x.reshape(-1).astype(jnp.int32)
    return _encoder_sc(xf, word_table, pos_table)


# speedup vs baseline: 3.9962x; 1.2870x over previous
import functools

import jax
import jax.numpy as jnp
from jax import lax
from jax.experimental import pallas as pl
from jax.experimental.pallas import tpu as pltpu
from jax.experimental.pallas import tpu_sc as plsc

B, L, D = 4096, 200, 64
NC, NS = 2, 16
NW = NC * NS
SEQ_PER_W = B // NW
S0, S1 = 96, 104
LANES = 16


def _encoder_sc(xf, wt, pt):
    mesh = plsc.VectorSubcoreMesh(core_axis_name="c", subcore_axis_name="s")

    @functools.partial(
        pl.kernel,
        mesh=mesh,
        out_type=jax.ShapeDtypeStruct((B, L, D), jnp.float32),
        scratch_types=[
            pltpu.VMEM((SEQ_PER_W * L,), jnp.int32),
            pltpu.VMEM((L, D), jnp.float32),
            pltpu.VMEM((L, D), jnp.float32),
            pltpu.VMEM((L, D), jnp.float32),
            pltpu.SemaphoreType.DMA,
            pltpu.SemaphoreType.DMA,
            pltpu.SemaphoreType.DMA,
            pltpu.SemaphoreType.DMA,
        ],
        compiler_params=pltpu.CompilerParams(use_tc_tiling_on_sc=False),
    )
    def k(xf_hbm, wt_hbm, pt_hbm, out_hbm,
          idx_all, rows0, rows1, pos_v, gsem0, gsem1, ssem0, ssem1):
        wid = lax.axis_index("s") * NC + lax.axis_index("c")
        seq0 = wid * SEQ_PER_W
        rows = (rows0, rows1)
        gsem = (gsem0, gsem1)
        ssem = (ssem0, ssem1)

        pltpu.sync_copy(pt_hbm.at[pl.ds(0, L)], pos_v)
        pltpu.sync_copy(xf_hbm.at[pl.ds(seq0 * L, SEQ_PER_W * L)], idx_all)

        def fire_gather(g, b):
            base = g * L
            pltpu.async_copy(wt_hbm.at[idx_all.at[pl.ds(base, S0)]],
                             rows[b].at[pl.ds(0, S0)], gsem[b])
            pltpu.async_copy(wt_hbm.at[idx_all.at[pl.ds(base + S0, S1)]],
                             rows[b].at[pl.ds(S0, S1)], gsem[b])

        def wait_gather(b):
            pltpu.make_async_copy(wt_hbm.at[pl.ds(0, L)], rows[b], gsem[b]).wait()

        def fire_scatter(g, b):
            pltpu.async_copy(rows[b], out_hbm.at[seq0 + g], ssem[b])

        def wait_scatter(b):
            pltpu.make_async_copy(rows[b], out_hbm.at[0], ssem[b]).wait()

        def add_pos(b):
            def add_row(r, carry):
                for c in range(D // LANES):
                    sl = pl.ds(c * LANES, LANES)
                    rows[b][r, sl] = rows[b][r, sl] + pos_v[r, sl]
                return carry
            lax.fori_loop(0, L, add_row, 0)

        def process(g, b):
            wait_gather(b)
            add_pos(b)
            fire_scatter(g, b)

        fire_gather(0, 0)
        fire_gather(1, 1)
        process(0, 0)

        def pair(kk, carry):
            g1 = 2 * kk + 1
            wait_scatter(0)
            fire_gather(g1 + 1, 0)
            process(g1, 1)
            wait_scatter(1)
            fire_gather(g1 + 2, 1)
            process(g1 + 1, 0)
            return carry

        lax.fori_loop(0, (SEQ_PER_W - 2) // 2, pair, 0)

        process(SEQ_PER_W - 1, 1)
        wait_scatter(0)
        wait_scatter(1)

    return k(xf, wt, pt)


def kernel(x, word_table, pos_table):
    xf = x.reshape(-1).astype(jnp.int32)
    return _encoder_sc(xf, word_table, pos_table)
